# baseline (speedup 1.0000x reference)
import jax
import jax.numpy as jnp
from jax import lax
from jax.experimental import pallas as pl
from jax.experimental.pallas import tpu as pltpu
from jax.experimental.pallas import tpu_sc as plsc

N = 100000
F = 1600000
B = 16

NUM_CORES = 2
NUM_SUBCORES = 16
L = 16

FACES_PER_TILE = F // NUM_SUBCORES
T = 2000
NCHUNK = FACES_PER_TILE // T
TI = T // L


def _rsqrt_newton(q):
  qi = lax.bitcast_convert_type(q, jnp.int32)
  yi = jnp.int32(0x5F3759DF) - lax.shift_right_logical(qi, 1)
  y = lax.bitcast_convert_type(yi, jnp.float32)
  for _ in range(3):
    y = y * (1.5 - 0.5 * q * y * y)
  return y


def _body(t1x, t1y, t1z, t2x, t2y, t2z, bd1, bd2, fa, fb, out,
          tx, ty, tz, partials,
          idxA0, idxA1, idxA2, idxB0, idxB1, idxB2,
          dA0x, dA0y, dA0z, dA1x, dA1y, dA1z, dA2x, dA2y, dA2z,
          dB0x, dB0y, dB0z, dB1x, dB1y, dB1z, dB2x, dB2y, dB2z,
          bbA, bbB, bndv, acc0, acc1, acc2, acc3, acc4, acc5, acc6,
          res, bigbuf, semIA, semIB, semGA, semGB):
  accs = [acc0, acc1, acc2, acc3, acc4, acc5, acc6]
  bufA = (idxA0, idxA1, idxA2,
          dA0x, dA0y, dA0z, dA1x, dA1y, dA1z, dA2x, dA2y, dA2z)
  bufB = (idxB0, idxB1, idxB2,
          dB0x, dB0y, dB0z, dB1x, dB1y, dB1z, dB2x, dB2y, dB2z)
  c = lax.axis_index("c")
  s = lax.axis_index("s")

  @pl.when(jnp.logical_and(c == 0, s == 0))
  def _():
    pltpu.sync_copy(t1x, tx)
    pltpu.sync_copy(t1y, ty)
    pltpu.sync_copy(t1z, tz)

  @pl.when(jnp.logical_and(c == 1, s == 0))
  def _():
    pltpu.sync_copy(t2x, tx)
    pltpu.sync_copy(t2y, ty)
    pltpu.sync_copy(t2z, tz)

  @pl.when(c == 0)
  def _():
    pltpu.sync_copy(bd1, bndv)

  @pl.when(c == 1)
  def _():
    pltpu.sync_copy(bd2, bndv)

  plsc.subcore_barrier()

  bvec = bndv[pl.ds(0, L)]
  bnds = [bvec[j] for j in range(1, B)]

  zeros = jnp.zeros((L,), jnp.float32)
  for a in accs:
    for r in range(L):
      a[pl.ds(r * L, L)] = zeros

  lanes = lax.iota(jnp.int32, L)
  loff = lanes * L
  ones = jnp.ones((L,), jnp.float32)

  def fire_idx(k, buf, sem):
    base = s * FACES_PER_TILE + k * T

    @pl.when(c == 0)
    def _():
      for v in range(3):
        pltpu.async_copy(fa.at[pl.ds(v * F + base, T)], buf[v], sem)

    @pl.when(c == 1)
    def _():
      for v in range(3):
        pltpu.async_copy(fb.at[pl.ds(v * F + base, T)], buf[v], sem)

  def wait_idx(buf, sem):
    for dst in buf[:3]:
      pltpu.make_async_copy(fa.at[pl.ds(0, T)], dst, sem).wait()

  def fire_gathers(buf, sem):
    idx0, idx1, idx2 = buf[:3]
    gathers = [
        (tx, idx0, buf[3]), (ty, idx0, buf[4]), (tz, idx0, buf[5]),
        (tx, idx1, buf[6]), (ty, idx1, buf[7]), (tz, idx1, buf[8]),
        (tx, idx2, buf[9]), (ty, idx2, buf[10]), (tz, idx2, buf[11]),
    ]
    return [pltpu.async_copy(t.at[i], d, sem) for (t, i, d) in gathers]

  def wait_gathers(buf, sem):
    for d in buf[3:]:
      pltpu.make_async_copy(tx.at[buf[0]], d, sem).wait()

  def precompute_bb(buf, bbuf):
    idx0 = buf[0]

    def bb_body(i, carry):
      sl = pl.ds(i * L, L)
      i0 = idx0[sl]
      cs = [jnp.where(i0 >= bj, 1, 0).astype(jnp.int32) for bj in bnds]
      while len(cs) > 1:
        nxt = [cs[k] + cs[k + 1] for k in range(0, len(cs) - 1, 2)]
        if len(cs) % 2:
          nxt.append(cs[-1])
        cs = nxt
      bbuf[sl] = cs[0]
      return carry

    lax.fori_loop(0, TI, bb_body, 0)

  def compute_chunk(buf, bbuf):
    (d0x, d0y, d0z,
     d1x, d1y, d1z, d2x, d2y, d2z) = buf[3:]

    def vec_body(i, carry2):
      sl = pl.ds(i * L, L)
      v0x, v0y, v0z = d0x[sl], d0y[sl], d0z[sl]
      v1x, v1y, v1z = d1x[sl], d1y[sl], d1z[sl]
      v2x, v2y, v2z = d2x[sl], d2y[sl], d2z[sl]

      s1x = v1x - v0x
      s1y = v1y - v0y
      s1z = v1z - v0z
      s2x = v2x - v0x
      s2y = v2y - v0y
      s2z = v2z - v0z

      nx = s1y * s2z - s1z * s2y
      ny = s1z * s2x - s1x * s2z
      nz = s1x * s2y - s1y * s2x

      qq = nx * nx + ny * ny + nz * nz
      y = _rsqrt_newton(qq)
      y = jnp.where(qq < 1e-24, 0.0, y)
      inv = 1.0 / (qq * y + 1e-8)
      ux = nx * inv
      uy = ny * inv
      uz = nz * inv

      slot = loff + bbuf[sl]
      plsc.addupdate_scatter(acc0, [slot], ones)
      plsc.addupdate_scatter(acc1, [slot], ux)
      plsc.addupdate_scatter(acc2, [slot], uy)
      plsc.addupdate_scatter(acc3, [slot], uz)
      plsc.addupdate_scatter(acc4, [slot], ux * ux)
      plsc.addupdate_scatter(acc5, [slot], uy * uy)
      plsc.addupdate_scatter(acc6, [slot], uz * uz)
      return carry2

    lax.fori_loop(0, TI, vec_body, 0)

  NPAIR = NCHUNK // 2
  fire_idx(0, bufA, semIA)
  wait_idx(bufA, semIA)
  fire_gathers(bufA, semGA)
  precompute_bb(bufA, bbA)
  fire_idx(1, bufB, semIB)

  def pair_body(j, carry):
    not_last = j < NPAIR - 1
    wait_idx(bufB, semIB)
    fire_gathers(bufB, semGB)
    precompute_bb(bufB, bbB)
    wait_gathers(bufA, semGA)

    @pl.when(not_last)
    def _():
      fire_idx(2 * j + 2, bufA, semIA)

    compute_chunk(bufA, bbA)

    @pl.when(not_last)
    def _():
      wait_idx(bufA, semIA)
      fire_gathers(bufA, semGA)
      precompute_bb(bufA, bbA)

    wait_gathers(bufB, semGB)

    @pl.when(not_last)
    def _():
      fire_idx(2 * j + 3, bufB, semIB)

    compute_chunk(bufB, bbB)
    return carry

  lax.fori_loop(0, NPAIR, pair_body, 0)

  for q, a in enumerate(accs):
    tot = a[pl.ds(0, L)]
    for r in range(1, L):
      tot = tot + a[pl.ds(r * L, L)]
    res[pl.ds(q * L, L)] = tot
  res[pl.ds(7 * L, L)] = zeros
  pltpu.sync_copy(res, partials.at[pl.ds(s * 8 * B, 8 * B)])
  plsc.subcore_barrier()

  @pl.when(s == 0)
  def _():
    pltpu.sync_copy(partials, bigbuf)
    for q in range(8):
      tot = bigbuf[pl.ds(q * L, L)]
      for t in range(1, NUM_SUBCORES):
        tot = tot + bigbuf[pl.ds(t * 8 * B + q * L, L)]
      res[pl.ds(q * L, L)] = tot
    pltpu.sync_copy(res, out.at[pl.ds(c * 8 * B, 8 * B)])


@jax.jit
def kernel(x1, x2, b1, b2, f1, f2):
  fa = f1.astype(jnp.int32).reshape(-1)
  fb = f2.astype(jnp.int32).reshape(-1)
  js = jnp.arange(1, B, dtype=b1.dtype)
  bd1 = jnp.concatenate(
      [jnp.zeros((1,), jnp.int32), jnp.searchsorted(b1, js).astype(jnp.int32)])
  bd2 = jnp.concatenate(
      [jnp.zeros((1,), jnp.int32), jnp.searchsorted(b2, js).astype(jnp.int32)])

  mesh = plsc.VectorSubcoreMesh(
      core_axis_name="c", subcore_axis_name="s",
      num_cores=NUM_CORES, num_subcores=NUM_SUBCORES)
  run = pl.kernel(
      _body,
      out_type=jax.ShapeDtypeStruct((2 * 8 * B,), jnp.float32),
      mesh=mesh,
      compiler_params=pltpu.CompilerParams(needs_layout_passes=False),
      scratch_types=[
          pltpu.VMEM_SHARED((N,), jnp.float32),
          pltpu.VMEM_SHARED((N,), jnp.float32),
          pltpu.VMEM_SHARED((N,), jnp.float32),
          pltpu.VMEM_SHARED((NUM_SUBCORES * 8 * B,), jnp.float32),
          pltpu.VMEM((T,), jnp.int32),
          pltpu.VMEM((T,), jnp.int32),
          pltpu.VMEM((T,), jnp.int32),
          pltpu.VMEM((T,), jnp.int32),
          pltpu.VMEM((T,), jnp.int32),
          pltpu.VMEM((T,), jnp.int32),
          pltpu.VMEM((T,), jnp.float32),
          pltpu.VMEM((T,), jnp.float32),
          pltpu.VMEM((T,), jnp.float32),
          pltpu.VMEM((T,), jnp.float32),
          pltpu.VMEM((T,), jnp.float32),
          pltpu.VMEM((T,), jnp.float32),
          pltpu.VMEM((T,), jnp.float32),
          pltpu.VMEM((T,), jnp.float32),
          pltpu.VMEM((T,), jnp.float32),
          pltpu.VMEM((T,), jnp.float32),
          pltpu.VMEM((T,), jnp.float32),
          pltpu.VMEM((T,), jnp.float32),
          pltpu.VMEM((T,), jnp.float32),
          pltpu.VMEM((T,), jnp.float32),
          pltpu.VMEM((T,), jnp.float32),
          pltpu.VMEM((T,), jnp.float32),
          pltpu.VMEM((T,), jnp.float32),
          pltpu.VMEM((T,), jnp.float32),
          pltpu.VMEM((T,), jnp.int32),
          pltpu.VMEM((T,), jnp.int32),
          pltpu.VMEM((B,), jnp.int32),
          pltpu.VMEM((L * B,), jnp.float32),
          pltpu.VMEM((L * B,), jnp.float32),
          pltpu.VMEM((L * B,), jnp.float32),
          pltpu.VMEM((L * B,), jnp.float32),
          pltpu.VMEM((L * B,), jnp.float32),
          pltpu.VMEM((L * B,), jnp.float32),
          pltpu.VMEM((L * B,), jnp.float32),
          pltpu.VMEM((8 * B,), jnp.float32),
          pltpu.VMEM((NUM_SUBCORES * 8 * B,), jnp.float32),
          pltpu.SemaphoreType.DMA,
          pltpu.SemaphoreType.DMA,
          pltpu.SemaphoreType.DMA,
          pltpu.SemaphoreType.DMA,
      ],
  )
  stats = run(x1[:, 0], x1[:, 1], x1[:, 2],
              x2[:, 0], x2[:, 1], x2[:, 2],
              bd1, bd2, fa, fb)
  stats = stats.reshape(2, 8, B)

  cnt1 = stats[0, 0]
  sum1 = stats[0, 1:4]
  cnt2 = stats[1, 0]
  sum2 = stats[1, 1:4]
  ssq2 = stats[1, 4:7]
  mean1 = sum1 / cnt1
  mean2 = sum2 / cnt2
  var2 = (ssq2 - cnt2 * mean2 * mean2) / (cnt2 - 1.0)
  std2 = jnp.sqrt(jnp.maximum(var2, 0.0))
  consistency = jnp.sum(jnp.sqrt(jnp.sum(std2 * std2, axis=0)))
  similarity = jnp.sum(jnp.sqrt(jnp.sum((mean1 - mean2) ** 2, axis=0)))
  return jnp.reshape(consistency + similarity, (1,))


# speedup vs baseline: 1.0864x; 1.0864x over previous
import jax
import jax.numpy as jnp
from jax import lax
from jax.experimental import pallas as pl
from jax.experimental.pallas import tpu as pltpu
from jax.experimental.pallas import tpu_sc as plsc

N = 100000
F = 1600000
B = 16

NUM_CORES = 2
NUM_SUBCORES = 16
L = 16

FACES_PER_TILE = F // NUM_SUBCORES
T = 2000
NCHUNK = FACES_PER_TILE // T
TI = T // L


def _rsqrt_newton(q):
  qi = lax.bitcast_convert_type(q, jnp.int32)
  yi = jnp.int32(0x5F3759DF) - lax.shift_right_logical(qi, 1)
  y = lax.bitcast_convert_type(yi, jnp.float32)
  for _ in range(3):
    y = y * (1.5 - 0.5 * q * y * y)
  return y


def _body(t1x, t1y, t1z, t1b, t2x, t2y, t2z, t2b,
          fa, fb, out,
          tx, ty, tz, tb, partials,
          idxA0, idxA1, idxA2, idxB0, idxB1, idxB2,
          dA0x, dA0y, dA0z, dA0b, dA1x, dA1y, dA1z, dA2x, dA2y, dA2z,
          dB0x, dB0y, dB0z, dB0b, dB1x, dB1y, dB1z, dB2x, dB2y, dB2z,
          acc0, acc1, acc2, acc3, acc4, acc5, acc6, res, bigbuf,
          semIA, semIB, semGA, semGB):
  accs = [acc0, acc1, acc2, acc3, acc4, acc5, acc6]
  bufA = (idxA0, idxA1, idxA2,
          dA0x, dA0y, dA0z, dA0b, dA1x, dA1y, dA1z, dA2x, dA2y, dA2z)
  bufB = (idxB0, idxB1, idxB2,
          dB0x, dB0y, dB0z, dB0b, dB1x, dB1y, dB1z, dB2x, dB2y, dB2z)
  c = lax.axis_index("c")
  s = lax.axis_index("s")

  @pl.when(jnp.logical_and(c == 0, s == 0))
  def _():
    pltpu.sync_copy(t1x, tx)
    pltpu.sync_copy(t1y, ty)
    pltpu.sync_copy(t1z, tz)
    pltpu.sync_copy(t1b, tb)

  @pl.when(jnp.logical_and(c == 1, s == 0))
  def _():
    pltpu.sync_copy(t2x, tx)
    pltpu.sync_copy(t2y, ty)
    pltpu.sync_copy(t2z, tz)
    pltpu.sync_copy(t2b, tb)

  plsc.subcore_barrier()

  zeros = jnp.zeros((L,), jnp.float32)
  for a in accs:
    for r in range(L):
      a[pl.ds(r * L, L)] = zeros

  lanes = lax.iota(jnp.int32, L)
  loff = lanes * L
  ones = jnp.ones((L,), jnp.float32)

  def fire_idx(k, buf, sem):
    base = s * FACES_PER_TILE + k * T

    @pl.when(c == 0)
    def _():
      for v in range(3):
        pltpu.async_copy(fa.at[pl.ds(v * F + base, T)], buf[v], sem)

    @pl.when(c == 1)
    def _():
      for v in range(3):
        pltpu.async_copy(fb.at[pl.ds(v * F + base, T)], buf[v], sem)

  def wait_idx(buf, sem):
    for dst in buf[:3]:
      pltpu.make_async_copy(fa.at[pl.ds(0, T)], dst, sem).wait()

  def fire_gathers(buf, sem):
    idx0, idx1, idx2 = buf[:3]
    gathers = [
        (tx, idx0, buf[3]), (ty, idx0, buf[4]), (tz, idx0, buf[5]),
        (tb, idx0, buf[6]),
        (tx, idx1, buf[7]), (ty, idx1, buf[8]), (tz, idx1, buf[9]),
        (tx, idx2, buf[10]), (ty, idx2, buf[11]), (tz, idx2, buf[12]),
    ]
    return [pltpu.async_copy(t.at[i], d, sem) for (t, i, d) in gathers]

  def wait_gathers(buf, sem):
    for d in buf[3:]:
      pltpu.make_async_copy(tx.at[buf[0]], d, sem).wait()

  def compute_chunk(buf):
    (_, _, _, d0x, d0y, d0z, d0b,
     d1x, d1y, d1z, d2x, d2y, d2z) = buf

    def vec_body(i, carry2):
      sl = pl.ds(i * L, L)
      v0x, v0y, v0z = d0x[sl], d0y[sl], d0z[sl]
      v1x, v1y, v1z = d1x[sl], d1y[sl], d1z[sl]
      v2x, v2y, v2z = d2x[sl], d2y[sl], d2z[sl]

      s1x = v1x - v0x
      s1y = v1y - v0y
      s1z = v1z - v0z
      s2x = v2x - v0x
      s2y = v2y - v0y
      s2z = v2z - v0z

      nx = s1y * s2z - s1z * s2y
      ny = s1z * s2x - s1x * s2z
      nz = s1x * s2y - s1y * s2x

      qq = nx * nx + ny * ny + nz * nz
      y = _rsqrt_newton(qq)
      inv = 1.0 / (qq * y + 1e-8)
      ux = nx * inv
      uy = ny * inv
      uz = nz * inv

      slot = loff + d0b[sl].astype(jnp.int32)
      plsc.addupdate_scatter(acc0, [slot], ones)
      plsc.addupdate_scatter(acc1, [slot], ux)
      plsc.addupdate_scatter(acc2, [slot], uy)
      plsc.addupdate_scatter(acc3, [slot], uz)
      plsc.addupdate_scatter(acc4, [slot], ux * ux)
      plsc.addupdate_scatter(acc5, [slot], uy * uy)
      plsc.addupdate_scatter(acc6, [slot], uz * uz)
      return carry2

    lax.fori_loop(0, TI, vec_body, 0)

  NPAIR = NCHUNK // 2
  fire_idx(0, bufA, semIA)
  wait_idx(bufA, semIA)
  fire_gathers(bufA, semGA)
  fire_idx(1, bufB, semIB)

  def pair_body(j, carry):
    not_last = j < NPAIR - 1
    wait_idx(bufB, semIB)
    fire_gathers(bufB, semGB)
    wait_gathers(bufA, semGA)

    @pl.when(not_last)
    def _():
      fire_idx(2 * j + 2, bufA, semIA)

    compute_chunk(bufA)

    @pl.when(not_last)
    def _():
      wait_idx(bufA, semIA)
      fire_gathers(bufA, semGA)

    wait_gathers(bufB, semGB)

    @pl.when(not_last)
    def _():
      fire_idx(2 * j + 3, bufB, semIB)

    compute_chunk(bufB)
    return carry

  lax.fori_loop(0, NPAIR, pair_body, 0)

  for q, a in enumerate(accs):
    tot = a[pl.ds(0, L)]
    for r in range(1, L):
      tot = tot + a[pl.ds(r * L, L)]
    res[pl.ds(q * L, L)] = tot
  res[pl.ds(7 * L, L)] = zeros
  pltpu.sync_copy(res, partials.at[pl.ds(s * 8 * B, 8 * B)])
  plsc.subcore_barrier()

  @pl.when(s == 0)
  def _():
    pltpu.sync_copy(partials, bigbuf)
    for q in range(8):
      tot = bigbuf[pl.ds(q * L, L)]
      for t in range(1, NUM_SUBCORES):
        tot = tot + bigbuf[pl.ds(t * 8 * B + q * L, L)]
      res[pl.ds(q * L, L)] = tot
    pltpu.sync_copy(res, out.at[pl.ds(c * 8 * B, 8 * B)])


@jax.jit
def kernel(x1, x2, b1, b2, f1, f2):
  t1b = b1.astype(jnp.float32)
  t2b = b2.astype(jnp.float32)
  fa = f1.astype(jnp.int32).reshape(-1)
  fb = f2.astype(jnp.int32).reshape(-1)

  mesh = plsc.VectorSubcoreMesh(
      core_axis_name="c", subcore_axis_name="s",
      num_cores=NUM_CORES, num_subcores=NUM_SUBCORES)
  run = pl.kernel(
      _body,
      out_type=jax.ShapeDtypeStruct((2 * 8 * B,), jnp.float32),
      mesh=mesh,
      compiler_params=pltpu.CompilerParams(needs_layout_passes=False),
      scratch_types=[
          pltpu.VMEM_SHARED((N,), jnp.float32),
          pltpu.VMEM_SHARED((N,), jnp.float32),
          pltpu.VMEM_SHARED((N,), jnp.float32),
          pltpu.VMEM_SHARED((N,), jnp.float32),
          pltpu.VMEM_SHARED((NUM_SUBCORES * 8 * B,), jnp.float32),
          pltpu.VMEM((T,), jnp.int32),
          pltpu.VMEM((T,), jnp.int32),
          pltpu.VMEM((T,), jnp.int32),
          pltpu.VMEM((T,), jnp.int32),
          pltpu.VMEM((T,), jnp.int32),
          pltpu.VMEM((T,), jnp.int32),
          pltpu.VMEM((T,), jnp.float32),
          pltpu.VMEM((T,), jnp.float32),
          pltpu.VMEM((T,), jnp.float32),
          pltpu.VMEM((T,), jnp.float32),
          pltpu.VMEM((T,), jnp.float32),
          pltpu.VMEM((T,), jnp.float32),
          pltpu.VMEM((T,), jnp.float32),
          pltpu.VMEM((T,), jnp.float32),
          pltpu.VMEM((T,), jnp.float32),
          pltpu.VMEM((T,), jnp.float32),
          pltpu.VMEM((T,), jnp.float32),
          pltpu.VMEM((T,), jnp.float32),
          pltpu.VMEM((T,), jnp.float32),
          pltpu.VMEM((T,), jnp.float32),
          pltpu.VMEM((T,), jnp.float32),
          pltpu.VMEM((T,), jnp.float32),
          pltpu.VMEM((T,), jnp.float32),
          pltpu.VMEM((T,), jnp.float32),
          pltpu.VMEM((T,), jnp.float32),
          pltpu.VMEM((T,), jnp.float32),
          pltpu.VMEM((L * B,), jnp.float32),
          pltpu.VMEM((L * B,), jnp.float32),
          pltpu.VMEM((L * B,), jnp.float32),
          pltpu.VMEM((L * B,), jnp.float32),
          pltpu.VMEM((L * B,), jnp.float32),
          pltpu.VMEM((L * B,), jnp.float32),
          pltpu.VMEM((L * B,), jnp.float32),
          pltpu.VMEM((8 * B,), jnp.float32),
          pltpu.VMEM((NUM_SUBCORES * 8 * B,), jnp.float32),
          pltpu.SemaphoreType.DMA,
          pltpu.SemaphoreType.DMA,
          pltpu.SemaphoreType.DMA,
          pltpu.SemaphoreType.DMA,
      ],
  )
  stats = run(x1[:, 0], x1[:, 1], x1[:, 2], t1b,
              x2[:, 0], x2[:, 1], x2[:, 2], t2b,
              fa, fb)
  stats = stats.reshape(2, 8, B)

  cnt1 = stats[0, 0]
  sum1 = stats[0, 1:4]
  cnt2 = stats[1, 0]
  sum2 = stats[1, 1:4]
  ssq2 = stats[1, 4:7]
  mean1 = sum1 / cnt1
  mean2 = sum2 / cnt2
  var2 = (ssq2 - cnt2 * mean2 * mean2) / (cnt2 - 1.0)
  std2 = jnp.sqrt(jnp.maximum(var2, 0.0))
  consistency = jnp.sum(jnp.sqrt(jnp.sum(std2 * std2, axis=0)))
  similarity = jnp.sum(jnp.sqrt(jnp.sum((mean1 - mean2) ** 2, axis=0)))
  return jnp.reshape(consistency + similarity, (1,))


# speedup vs baseline: 2.2642x; 2.0841x over previous
import jax
import jax.numpy as jnp
from jax import lax
from jax.experimental import pallas as pl
from jax.experimental.pallas import tpu as pltpu
from jax.experimental.pallas import tpu_sc as plsc

N = 100000
F = 1600000
B = 16

NUM_CORES = 2
NUM_SUBCORES = 16
L = 16

FACES_PER_TILE = F // NUM_SUBCORES
T = 2000
NCHUNK = FACES_PER_TILE // T
TI = T // L


def _rsqrt_newton(q):
  qi = lax.bitcast_convert_type(q, jnp.int32)
  yi = jnp.int32(0x5F3759DF) - lax.shift_right_logical(qi, 1)
  y = lax.bitcast_convert_type(yi, jnp.float32)
  for _ in range(3):
    y = y * (1.5 - 0.5 * q * y * y)
  return y


def _body(t1x, t1y, t1z, t1b, t2x, t2y, t2z, t2b,
          fa0, fa1, fa2, fb0, fb1, fb2, out,
          tx, ty, tz, tb, partials,
          idxA0, idxA1, idxA2, idxB0, idxB1, idxB2,
          dA0x, dA0y, dA0z, dA0b, dA1x, dA1y, dA1z, dA2x, dA2y, dA2z,
          dB0x, dB0y, dB0z, dB0b, dB1x, dB1y, dB1z, dB2x, dB2y, dB2z,
          acc0, acc1, acc2, acc3, acc4, acc5, acc6, res, bigbuf,
          semIA, semIB, semGA, semGB):
  accs = [acc0, acc1, acc2, acc3, acc4, acc5, acc6]
  bufA = (idxA0, idxA1, idxA2,
          dA0x, dA0y, dA0z, dA0b, dA1x, dA1y, dA1z, dA2x, dA2y, dA2z)
  bufB = (idxB0, idxB1, idxB2,
          dB0x, dB0y, dB0z, dB0b, dB1x, dB1y, dB1z, dB2x, dB2y, dB2z)
  c = lax.axis_index("c")
  s = lax.axis_index("s")

  @pl.when(jnp.logical_and(c == 0, s == 0))
  def _():
    pltpu.sync_copy(t1x, tx)
    pltpu.sync_copy(t1y, ty)
    pltpu.sync_copy(t1z, tz)
    pltpu.sync_copy(t1b, tb)

  @pl.when(jnp.logical_and(c == 1, s == 0))
  def _():
    pltpu.sync_copy(t2x, tx)
    pltpu.sync_copy(t2y, ty)
    pltpu.sync_copy(t2z, tz)
    pltpu.sync_copy(t2b, tb)

  plsc.subcore_barrier()

  zeros = jnp.zeros((L,), jnp.float32)
  for a in accs:
    for r in range(L):
      a[pl.ds(r * L, L)] = zeros

  lanes = lax.iota(jnp.int32, L)
  loff = lanes * L
  ones = jnp.ones((L,), jnp.float32)

  def fire_idx(k, buf, sem):
    base = s * FACES_PER_TILE + k * T

    @pl.when(c == 0)
    def _():
      for src, dst in ((fa0, buf[0]), (fa1, buf[1]), (fa2, buf[2])):
        pltpu.async_copy(src.at[pl.ds(base, T)], dst, sem)

    @pl.when(c == 1)
    def _():
      for src, dst in ((fb0, buf[0]), (fb1, buf[1]), (fb2, buf[2])):
        pltpu.async_copy(src.at[pl.ds(base, T)], dst, sem)

  def wait_idx(buf, sem):
    for dst in buf[:3]:
      pltpu.make_async_copy(fa0.at[pl.ds(0, T)], dst, sem).wait()

  def fire_gathers(buf, sem):
    idx0, idx1, idx2 = buf[:3]
    gathers = [
        (tx, idx0, buf[3]), (ty, idx0, buf[4]), (tz, idx0, buf[5]),
        (tb, idx0, buf[6]),
        (tx, idx1, buf[7]), (ty, idx1, buf[8]), (tz, idx1, buf[9]),
        (tx, idx2, buf[10]), (ty, idx2, buf[11]), (tz, idx2, buf[12]),
    ]
    return [pltpu.async_copy(t.at[i], d, sem) for (t, i, d) in gathers]

  def wait_gathers(buf, sem):
    for d in buf[3:]:
      pltpu.make_async_copy(tx.at[buf[0]], d, sem).wait()

  def compute_chunk(buf):
    (_, _, _, d0x, d0y, d0z, d0b,
     d1x, d1y, d1z, d2x, d2y, d2z) = buf

    def vec_body(i, carry2):
      sl = pl.ds(i * L, L)
      v0x, v0y, v0z = d0x[sl], d0y[sl], d0z[sl]
      v1x, v1y, v1z = d1x[sl], d1y[sl], d1z[sl]
      v2x, v2y, v2z = d2x[sl], d2y[sl], d2z[sl]

      s1x = v1x - v0x
      s1y = v1y - v0y
      s1z = v1z - v0z
      s2x = v2x - v0x
      s2y = v2y - v0y
      s2z = v2z - v0z

      nx = s1y * s2z - s1z * s2y
      ny = s1z * s2x - s1x * s2z
      nz = s1x * s2y - s1y * s2x

      qq = nx * nx + ny * ny + nz * nz
      y = _rsqrt_newton(qq)
      inv = 1.0 / (qq * y + 1e-8)
      ux = nx * inv
      uy = ny * inv
      uz = nz * inv

      slot = loff + d0b[sl].astype(jnp.int32)
      plsc.addupdate_scatter(acc0, [slot], ones)
      plsc.addupdate_scatter(acc1, [slot], ux)
      plsc.addupdate_scatter(acc2, [slot], uy)
      plsc.addupdate_scatter(acc3, [slot], uz)
      plsc.addupdate_scatter(acc4, [slot], ux * ux)
      plsc.addupdate_scatter(acc5, [slot], uy * uy)
      plsc.addupdate_scatter(acc6, [slot], uz * uz)
      return carry2

    lax.fori_loop(0, TI, vec_body, 0)

  NPAIR = NCHUNK // 2
  fire_idx(0, bufA, semIA)
  wait_idx(bufA, semIA)
  fire_gathers(bufA, semGA)
  fire_idx(1, bufB, semIB)

  def pair_body(j, carry):
    not_last = j < NPAIR - 1
    wait_idx(bufB, semIB)
    fire_gathers(bufB, semGB)
    wait_gathers(bufA, semGA)

    @pl.when(not_last)
    def _():
      fire_idx(2 * j + 2, bufA, semIA)

    compute_chunk(bufA)

    @pl.when(not_last)
    def _():
      wait_idx(bufA, semIA)
      fire_gathers(bufA, semGA)

    wait_gathers(bufB, semGB)

    @pl.when(not_last)
    def _():
      fire_idx(2 * j + 3, bufB, semIB)

    compute_chunk(bufB)
    return carry

  lax.fori_loop(0, NPAIR, pair_body, 0)

  for q, a in enumerate(accs):
    tot = a[pl.ds(0, L)]
    for r in range(1, L):
      tot = tot + a[pl.ds(r * L, L)]
    res[pl.ds(q * L, L)] = tot
  res[pl.ds(7 * L, L)] = zeros
  pltpu.sync_copy(res, partials.at[pl.ds(s * 8 * B, 8 * B)])
  plsc.subcore_barrier()

  @pl.when(s == 0)
  def _():
    pltpu.sync_copy(partials, bigbuf)
    for q in range(8):
      tot = bigbuf[pl.ds(q * L, L)]
      for t in range(1, NUM_SUBCORES):
        tot = tot + bigbuf[pl.ds(t * 8 * B + q * L, L)]
      res[pl.ds(q * L, L)] = tot
    pltpu.sync_copy(res, out.at[pl.ds(c * 8 * B, 8 * B)])


@jax.jit
def kernel(x1, x2, b1, b2, f1, f2):
  t1b = b1.astype(jnp.float32)
  t2b = b2.astype(jnp.float32)
  f1i = f1.astype(jnp.int32)
  f2i = f2.astype(jnp.int32)

  mesh = plsc.VectorSubcoreMesh(
      core_axis_name="c", subcore_axis_name="s",
      num_cores=NUM_CORES, num_subcores=NUM_SUBCORES)
  run = pl.kernel(
      _body,
      out_type=jax.ShapeDtypeStruct((2 * 8 * B,), jnp.float32),
      mesh=mesh,
      compiler_params=pltpu.CompilerParams(needs_layout_passes=False),
      scratch_types=[
          pltpu.VMEM_SHARED((N,), jnp.float32),
          pltpu.VMEM_SHARED((N,), jnp.float32),
          pltpu.VMEM_SHARED((N,), jnp.float32),
          pltpu.VMEM_SHARED((N,), jnp.float32),
          pltpu.VMEM_SHARED((NUM_SUBCORES * 8 * B,), jnp.float32),
          pltpu.VMEM((T,), jnp.int32),
          pltpu.VMEM((T,), jnp.int32),
          pltpu.VMEM((T,), jnp.int32),
          pltpu.VMEM((T,), jnp.int32),
          pltpu.VMEM((T,), jnp.int32),
          pltpu.VMEM((T,), jnp.int32),
          pltpu.VMEM((T,), jnp.float32),
          pltpu.VMEM((T,), jnp.float32),
          pltpu.VMEM((T,), jnp.float32),
          pltpu.VMEM((T,), jnp.float32),
          pltpu.VMEM((T,), jnp.float32),
          pltpu.VMEM((T,), jnp.float32),
          pltpu.VMEM((T,), jnp.float32),
          pltpu.VMEM((T,), jnp.float32),
          pltpu.VMEM((T,), jnp.float32),
          pltpu.VMEM((T,), jnp.float32),
          pltpu.VMEM((T,), jnp.float32),
          pltpu.VMEM((T,), jnp.float32),
          pltpu.VMEM((T,), jnp.float32),
          pltpu.VMEM((T,), jnp.float32),
          pltpu.VMEM((T,), jnp.float32),
          pltpu.VMEM((T,), jnp.float32),
          pltpu.VMEM((T,), jnp.float32),
          pltpu.VMEM((T,), jnp.float32),
          pltpu.VMEM((T,), jnp.float32),
          pltpu.VMEM((T,), jnp.float32),
          pltpu.VMEM((L * B,), jnp.float32),
          pltpu.VMEM((L * B,), jnp.float32),
          pltpu.VMEM((L * B,), jnp.float32),
          pltpu.VMEM((L * B,), jnp.float32),
          pltpu.VMEM((L * B,), jnp.float32),
          pltpu.VMEM((L * B,), jnp.float32),
          pltpu.VMEM((L * B,), jnp.float32),
          pltpu.VMEM((8 * B,), jnp.float32),
          pltpu.VMEM((NUM_SUBCORES * 8 * B,), jnp.float32),
          pltpu.SemaphoreType.DMA,
          pltpu.SemaphoreType.DMA,
          pltpu.SemaphoreType.DMA,
          pltpu.SemaphoreType.DMA,
      ],
  )
  stats = run(x1[:, 0], x1[:, 1], x1[:, 2], t1b,
              x2[:, 0], x2[:, 1], x2[:, 2], t2b,
              f1i[0], f1i[1], f1i[2], f2i[0], f2i[1], f2i[2])
  stats = stats.reshape(2, 8, B)

  cnt1 = stats[0, 0]
  sum1 = stats[0, 1:4]
  cnt2 = stats[1, 0]
  sum2 = stats[1, 1:4]
  ssq2 = stats[1, 4:7]
  mean1 = sum1 / cnt1
  mean2 = sum2 / cnt2
  var2 = (ssq2 - cnt2 * mean2 * mean2) / (cnt2 - 1.0)
  std2 = jnp.sqrt(jnp.maximum(var2, 0.0))
  consistency = jnp.sum(jnp.sqrt(jnp.sum(std2 * std2, axis=0)))
  similarity = jnp.sum(jnp.sqrt(jnp.sum((mean1 - mean2) ** 2, axis=0)))
  return jnp.reshape(consistency + similarity, (1,))
